# baseline (speedup 1.0000x reference)
import functools
import numpy as np

import jax
import jax.numpy as jnp
from jax import lax
from jax.experimental import pallas as pl
from jax.experimental.pallas import tpu as pltpu
from jax.experimental.pallas import tpu_sc as plsc

N = 10000
NPAD = 10240
E = 320000
D_IN = 128
H = 256
HH = 128
C = 32
G = 128
ALPHA = 0.5
THETA = 0.1
BLK = 512
NBLK = NPAD // BLK


def _lin1_body(cnt0, cnt1, x, w1, b1, x0_out, ht0_out, ht1_out, dinv_out):
    deg = cnt0[...] + cnt1[...] + 1.0
    dinv = lax.rsqrt(deg)
    h = jnp.dot(x[...], w1[...], preferred_element_type=jnp.float32) + b1[...]
    h = jnp.maximum(h, 0.0)
    x0_out[...] = h
    ht = dinv * h
    ht0_out[...] = ht[:, :HH]
    ht1_out[...] = ht[:, HH:]
    dinv_out[...] = dinv


def _lin1_stage(cnt0, cnt1, x_pad, w1, b1):
    return pl.pallas_call(
        _lin1_body,
        grid=(NBLK,),
        in_specs=[
            pl.BlockSpec((BLK, 1), lambda i: (i, 0)),
            pl.BlockSpec((BLK, 1), lambda i: (i, 0)),
            pl.BlockSpec((BLK, D_IN), lambda i: (i, 0)),
            pl.BlockSpec((D_IN, H), lambda i: (0, 0)),
            pl.BlockSpec((1, H), lambda i: (0, 0)),
        ],
        out_specs=[
            pl.BlockSpec((BLK, H), lambda i: (i, 0)),
            pl.BlockSpec((BLK, HH), lambda i: (i, 0)),
            pl.BlockSpec((BLK, HH), lambda i: (i, 0)),
            pl.BlockSpec((BLK, 1), lambda i: (i, 0)),
        ],
        out_shape=[
            jax.ShapeDtypeStruct((NPAD, H), jnp.float32),
            jax.ShapeDtypeStruct((NPAD, HH), jnp.float32),
            jax.ShapeDtypeStruct((NPAD, HH), jnp.float32),
            jax.ShapeDtypeStruct((NPAD, 1), jnp.float32),
        ],
    )(cnt0, cnt1, x_pad, w1, b1)


def _layer_body(s0, s1, ht0, ht1, x0, dinv, w, h_out, ht0_out, ht1_out,
                *, beta, emit_halves):
    sfull = jnp.concatenate([s0[...] + ht0[...], s1[...] + ht1[...]], axis=1)
    dv = dinv[...]
    agg = dv * sfull
    out = (1.0 - ALPHA) * agg + ALPHA * x0[...]
    z = (1.0 - beta) * out + beta * jnp.dot(
        out, w[...], preferred_element_type=jnp.float32)
    h = jnp.maximum(z, 0.0)
    h_out[...] = h
    if emit_halves:
        ht = dv * h
        ht0_out[...] = ht[:, :HH]
        ht1_out[...] = ht[:, HH:]


def _layer_stage(s0, s1, ht0, ht1, x0, dinv, w, beta, emit_halves):
    half = pl.BlockSpec((BLK, HH), lambda i: (i, 0))
    outs = [pl.BlockSpec((BLK, H), lambda i: (i, 0))]
    out_shapes = [jax.ShapeDtypeStruct((NPAD, H), jnp.float32)]
    if emit_halves:
        outs += [half, half]
        out_shapes += [jax.ShapeDtypeStruct((NPAD, HH), jnp.float32)] * 2
    body = functools.partial(_layer_body, beta=beta, emit_halves=emit_halves)
    if not emit_halves:
        def body(s0, s1, ht0, ht1, x0, dinv, w, h_out):
            _layer_body(s0, s1, ht0, ht1, x0, dinv, w, h_out, None, None,
                        beta=beta, emit_halves=False)
    return pl.pallas_call(
        body,
        grid=(NBLK,),
        in_specs=[half, half, half, half,
                  pl.BlockSpec((BLK, H), lambda i: (i, 0)),
                  pl.BlockSpec((BLK, 1), lambda i: (i, 0)),
                  pl.BlockSpec((H, H), lambda i: (0, 0))],
        out_specs=outs,
        out_shape=out_shapes,
    )(s0, s1, ht0, ht1, x0, dinv, w)


def _pool_body(h, batch, w2, b2, out, acc, cnt):
    i = pl.program_id(0)

    @pl.when(i == 0)
    def _():
        acc[...] = jnp.zeros_like(acc)
        cnt[...] = jnp.zeros_like(cnt)

    gid = lax.broadcasted_iota(jnp.int32, (BLK, G), 1)
    p = (batch[...] == gid).astype(jnp.float32)
    dn = (((0,), (0,)), ((), ()))
    acc[...] += lax.dot_general(p, h[...], dn,
                                preferred_element_type=jnp.float32)
    cnt[...] += lax.dot_general(p, jnp.ones((BLK, 1), jnp.float32), dn,
                                preferred_element_type=jnp.float32)

    @pl.when(i == NBLK - 1)
    def _():
        pooled = acc[...] / jnp.maximum(cnt[...], 1.0)
        out[...] = jnp.dot(pooled, w2[...],
                           preferred_element_type=jnp.float32) + b2[...]


def _pool_stage(h3, batch2d, w2, b2):
    return pl.pallas_call(
        _pool_body,
        grid=(NBLK,),
        in_specs=[
            pl.BlockSpec((BLK, H), lambda i: (i, 0)),
            pl.BlockSpec((BLK, 1), lambda i: (i, 0)),
            pl.BlockSpec((H, C), lambda i: (0, 0)),
            pl.BlockSpec((1, C), lambda i: (0, 0)),
        ],
        out_specs=pl.BlockSpec((G, C), lambda i: (0, 0)),
        out_shape=jax.ShapeDtypeStruct((G, C), jnp.float32),
        scratch_shapes=[pltpu.VMEM((G, H), jnp.float32),
                        pltpu.VMEM((G, 1), jnp.float32)],
    )(h3, batch2d, w2, b2)


_MESH = plsc.VectorSubcoreMesh(core_axis_name="c", subcore_axis_name="s",
                               num_cores=2, num_subcores=16)
NTILE = 16
ECHUNK = 128
ROWS_PER_TILE = NPAD // NTILE

_CNT_EDGES = E // 32
_CNT_FULL = _CNT_EDGES // ECHUNK
_CNT_REM = _CNT_EDGES - _CNT_FULL * ECHUNK

_ECH = 128
_INNER = 16
_BLOCKS = 10
_AGG_CHUNKS = _INNER * _BLOCKS
EPAD = _AGG_CHUNKS * _ECH * NTILE
_NPADROWS = NPAD - N


def _zero_vmem_2d(ref, nrows):
    def zrow(i, _):
        for j in range(HH // 16):
            ref[i, pl.ds(j * 16, 16)] = jnp.zeros((16,), jnp.float32)
        return 0
    lax.fori_loop(0, nrows, zrow, 0)


def _count_body(col_hbm, out, cnt_sh, z640, col_v, ones_v, col16_v, ones16_v):
    c = lax.axis_index("c")
    s = lax.axis_index("s")
    for j in range(640 // 16):
        z640[pl.ds(j * 16, 16)] = jnp.zeros((16,), jnp.float32)
    for j in range(ECHUNK // 16):
        ones_v[pl.ds(j * 16, 16)] = jnp.ones((16,), jnp.float32)
    ones16_v[...] = jnp.ones((16,), jnp.float32)
    pltpu.sync_copy(z640, cnt_sh.at[pl.ds(s * ROWS_PER_TILE, ROWS_PER_TILE)])
    plsc.subcore_barrier()

    base0 = (c * NTILE + s) * _CNT_EDGES

    def step(i, _):
        base = base0 + i * ECHUNK
        pltpu.sync_copy(col_hbm.at[pl.ds(base, ECHUNK)], col_v)
        pltpu.sync_copy(ones_v, cnt_sh.at[col_v], add=True)
        return 0

    lax.fori_loop(0, _CNT_FULL, step, 0)
    base = base0 + _CNT_FULL * ECHUNK
    pltpu.sync_copy(col_hbm.at[pl.ds(base, _CNT_REM)], col16_v)
    pltpu.sync_copy(ones16_v, cnt_sh.at[col16_v], add=True)
    plsc.subcore_barrier()
    sl = pl.ds(s * ROWS_PER_TILE, ROWS_PER_TILE)
    pltpu.sync_copy(cnt_sh.at[sl], z640)
    pltpu.sync_copy(z640, out.at[c, sl])


def _count_stage(col):
    out = pl.kernel(
        _count_body,
        out_type=jax.ShapeDtypeStruct((2, NPAD), jnp.float32),
        mesh=_MESH,
        scratch_types=[
            pltpu.VMEM_SHARED((NPAD,), jnp.float32),
            pltpu.VMEM((ROWS_PER_TILE,), jnp.float32),
            pltpu.VMEM((ECHUNK,), jnp.int32),
            pltpu.VMEM((ECHUNK,), jnp.float32),
            pltpu.VMEM((16,), jnp.int32),
            pltpu.VMEM((16,), jnp.float32),
        ],
    )(col)
    return out[0].reshape(NPAD, 1), out[1].reshape(NPAD, 1)


def _agg_body(ht0, ht1, row2d, col2d, s0, s1, agg_sh, row_v, col_v,
              buf_a, buf_b, ga, gb, ta, tb):
    c = lax.axis_index("c")
    s = lax.axis_index("s")

    def fire_g(k, buf, sem):
        @pl.when(c == 0)
        def _():
            pltpu.async_copy(ht0.at[row_v.at[k]], buf, sem)

        @pl.when(c == 1)
        def _():
            pltpu.async_copy(ht1.at[row_v.at[k]], buf, sem)

    def wait_g(buf, sem):
        pltpu.make_async_copy(ht0.at[pl.ds(0, _ECH)], buf, sem).wait()

    def fire_s(k, buf, sem):
        pltpu.async_copy(buf, agg_sh.at[col_v.at[k]], sem, add=True)

    def wait_s(buf, sem):
        pltpu.make_async_copy(buf, agg_sh.at[pl.ds(0, _ECH)], sem).wait()

    _zero_vmem_2d(buf_a, _ECH)
    for k in range(ROWS_PER_TILE // _ECH):
        pltpu.sync_copy(
            buf_a, agg_sh.at[pl.ds(s * ROWS_PER_TILE + k * _ECH, _ECH)])
    plsc.subcore_barrier()

    def block(o, _):
        obase = s * _AGG_CHUNKS + o * _INNER
        pltpu.sync_copy(row2d.at[pl.ds(obase, _INNER)], row_v)
        pltpu.sync_copy(col2d.at[pl.ds(obase, _INNER)], col_v)
        fire_g(0, buf_a, ga)
        fire_g(1, buf_b, gb)

        def step(j, _):
            k0 = 2 * j
            wait_g(buf_a, ga)
            fire_s(k0, buf_a, ta)
            wait_g(buf_b, gb)
            fire_s(k0 + 1, buf_b, tb)
            wait_s(buf_a, ta)

            @pl.when(k0 + 2 < _INNER)
            def _():
                fire_g(k0 + 2, buf_a, ga)

            wait_s(buf_b, tb)

            @pl.when(k0 + 3 < _INNER)
            def _():
                fire_g(k0 + 3, buf_b, gb)
            return 0

        lax.fori_loop(0, _INNER // 2, step, 0)
        return 0

    lax.fori_loop(0, _BLOCKS, block, 0)
    plsc.subcore_barrier()

    for k in range(ROWS_PER_TILE // _ECH):
        sl = pl.ds(s * ROWS_PER_TILE + k * _ECH, _ECH)
        pltpu.sync_copy(agg_sh.at[sl], buf_a)

        @pl.when(c == 0)
        def _():
            pltpu.sync_copy(buf_a, s0.at[sl])

        @pl.when(c == 1)
        def _():
            pltpu.sync_copy(buf_a, s1.at[sl])


def _agg_stage(ht0, ht1, row2d, col2d):
    return pl.kernel(
        _agg_body,
        out_type=[jax.ShapeDtypeStruct((NPAD, HH), jnp.float32),
                  jax.ShapeDtypeStruct((NPAD, HH), jnp.float32)],
        mesh=_MESH,
        scratch_types=[
            pltpu.VMEM_SHARED((NPAD, HH), jnp.float32),
            pltpu.VMEM((_INNER, _ECH), jnp.int32),
            pltpu.VMEM((_INNER, _ECH), jnp.int32),
            pltpu.VMEM((_ECH, HH), jnp.float32),
            pltpu.VMEM((_ECH, HH), jnp.float32),
            pltpu.SemaphoreType.DMA,
            pltpu.SemaphoreType.DMA,
            pltpu.SemaphoreType.DMA,
            pltpu.SemaphoreType.DMA,
        ],
    )(ht0, ht1, row2d, col2d)


def kernel(x, edge_index, batch, lin1_w, lin1_b, conv_w0, conv_w1, conv_w2,
           lin2_w, lin2_b):
    row = edge_index[0]
    col = edge_index[1]
    pad_idx = (N + jnp.arange(EPAD - E, dtype=jnp.int32) % _NPADROWS)
    row2d = jnp.concatenate([row, pad_idx]).reshape(EPAD // _ECH, _ECH)
    col2d = jnp.concatenate([col, pad_idx]).reshape(EPAD // _ECH, _ECH)
    x_pad = jnp.zeros((NPAD, D_IN), jnp.float32).at[:N].set(x)
    batch2d = jnp.full((NPAD, 1), G, jnp.int32).at[:N, 0].set(batch)
    b1 = lin1_b.reshape(1, H)
    b2 = lin2_b.reshape(1, C)

    cnt0, cnt1 = _count_stage(col)
    x0, ht0, ht1, dinv = _lin1_stage(cnt0, cnt1, x_pad, lin1_w, b1)

    betas = [float(np.log(THETA / l + 1.0)) for l in (1, 2, 3)]
    for li, (w, beta) in enumerate(zip((conv_w0, conv_w1, conv_w2), betas)):
        s0, s1 = _agg_stage(ht0, ht1, row2d, col2d)
        if li < 2:
            _, ht0, ht1 = _layer_stage(s0, s1, ht0, ht1, x0, dinv, w, beta,
                                       emit_halves=True)
        else:
            (h3,) = _layer_stage(s0, s1, ht0, ht1, x0, dinv, w, beta,
                                 emit_halves=False)

    return _pool_stage(h3, batch2d, lin2_w, b2)


# speedup vs baseline: 1.3204x; 1.3204x over previous
import functools
import numpy as np

import jax
import jax.numpy as jnp
from jax import lax
from jax.experimental import pallas as pl
from jax.experimental.pallas import tpu as pltpu
from jax.experimental.pallas import tpu_sc as plsc

N = 10000
NPAD = 10240
E = 320000
D_IN = 128
H = 256
HH = 128
C = 32
G = 128
ALPHA = 0.5
THETA = 0.1
BLK = 512
NBLK = NPAD // BLK


def _lin1_body(cnt0, cnt1, x, w1, b1, x0_out, ht0_out, ht1_out, dinv_out):
    deg = cnt0[...] + cnt1[...] + 1.0
    dinv = lax.rsqrt(deg)
    h = jnp.dot(x[...], w1[...], preferred_element_type=jnp.float32) + b1[...]
    h = jnp.maximum(h, 0.0)
    x0_out[...] = h
    ht = dinv * h
    ht0_out[...] = ht[:, :HH]
    ht1_out[...] = ht[:, HH:]
    dinv_out[...] = dinv


def _lin1_stage(cnt0, cnt1, x_pad, w1, b1):
    return pl.pallas_call(
        _lin1_body,
        grid=(NBLK,),
        in_specs=[
            pl.BlockSpec((BLK, 1), lambda i: (i, 0)),
            pl.BlockSpec((BLK, 1), lambda i: (i, 0)),
            pl.BlockSpec((BLK, D_IN), lambda i: (i, 0)),
            pl.BlockSpec((D_IN, H), lambda i: (0, 0)),
            pl.BlockSpec((1, H), lambda i: (0, 0)),
        ],
        out_specs=[
            pl.BlockSpec((BLK, H), lambda i: (i, 0)),
            pl.BlockSpec((BLK, HH), lambda i: (i, 0)),
            pl.BlockSpec((BLK, HH), lambda i: (i, 0)),
            pl.BlockSpec((BLK, 1), lambda i: (i, 0)),
        ],
        out_shape=[
            jax.ShapeDtypeStruct((NPAD, H), jnp.float32),
            jax.ShapeDtypeStruct((NPAD, HH), jnp.float32),
            jax.ShapeDtypeStruct((NPAD, HH), jnp.float32),
            jax.ShapeDtypeStruct((NPAD, 1), jnp.float32),
        ],
    )(cnt0, cnt1, x_pad, w1, b1)


def _layer_h(s0, s1, ht0, ht1, x0, dinv, w, beta):
    sfull = jnp.concatenate([s0[...] + ht0[...], s1[...] + ht1[...]], axis=1)
    agg = dinv[...] * sfull
    out = (1.0 - ALPHA) * agg + ALPHA * x0[...]
    z = (1.0 - beta) * out + beta * jnp.dot(
        out, w[...], preferred_element_type=jnp.float32)
    return jnp.maximum(z, 0.0)


def _layer_body(s0, s1, ht0, ht1, x0, dinv, w, ht0_out, ht1_out, *, beta):
    h = _layer_h(s0, s1, ht0, ht1, x0, dinv, w, beta)
    ht = dinv[...] * h
    ht0_out[...] = ht[:, :HH]
    ht1_out[...] = ht[:, HH:]


def _layer_stage(s0, s1, ht0, ht1, x0, dinv, w, beta):
    half = pl.BlockSpec((BLK, HH), lambda i: (i, 0))
    return pl.pallas_call(
        functools.partial(_layer_body, beta=beta),
        grid=(NBLK,),
        in_specs=[half, half, half, half,
                  pl.BlockSpec((BLK, H), lambda i: (i, 0)),
                  pl.BlockSpec((BLK, 1), lambda i: (i, 0)),
                  pl.BlockSpec((H, H), lambda i: (0, 0))],
        out_specs=[half, half],
        out_shape=[jax.ShapeDtypeStruct((NPAD, HH), jnp.float32)] * 2,
    )(s0, s1, ht0, ht1, x0, dinv, w)


def _layer3_pool_body(s0, s1, ht0, ht1, x0, dinv, w, batch, w2, b2, out,
                      acc, cnt, *, beta):
    i = pl.program_id(0)

    @pl.when(i == 0)
    def _():
        acc[...] = jnp.zeros_like(acc)
        cnt[...] = jnp.zeros_like(cnt)

    h = _layer_h(s0, s1, ht0, ht1, x0, dinv, w, beta)
    gid = lax.broadcasted_iota(jnp.int32, (BLK, G), 1)
    p = (batch[...] == gid).astype(jnp.float32)
    dn = (((0,), (0,)), ((), ()))
    acc[...] += lax.dot_general(p, h, dn, preferred_element_type=jnp.float32)
    cnt[...] += lax.dot_general(p, jnp.ones((BLK, 1), jnp.float32), dn,
                                preferred_element_type=jnp.float32)

    @pl.when(i == NBLK - 1)
    def _():
        pooled = acc[...] / jnp.maximum(cnt[...], 1.0)
        out[...] = jnp.dot(pooled, w2[...],
                           preferred_element_type=jnp.float32) + b2[...]


def _layer3_pool_stage(s0, s1, ht0, ht1, x0, dinv, w, beta, batch2d, w2, b2):
    half = pl.BlockSpec((BLK, HH), lambda i: (i, 0))
    return pl.pallas_call(
        functools.partial(_layer3_pool_body, beta=beta),
        grid=(NBLK,),
        in_specs=[half, half, half, half,
                  pl.BlockSpec((BLK, H), lambda i: (i, 0)),
                  pl.BlockSpec((BLK, 1), lambda i: (i, 0)),
                  pl.BlockSpec((H, H), lambda i: (0, 0)),
                  pl.BlockSpec((BLK, 1), lambda i: (i, 0)),
                  pl.BlockSpec((H, C), lambda i: (0, 0)),
                  pl.BlockSpec((1, C), lambda i: (0, 0))],
        out_specs=pl.BlockSpec((G, C), lambda i: (0, 0)),
        out_shape=jax.ShapeDtypeStruct((G, C), jnp.float32),
        scratch_shapes=[pltpu.VMEM((G, H), jnp.float32),
                        pltpu.VMEM((G, 1), jnp.float32)],
    )(s0, s1, ht0, ht1, x0, dinv, w, batch2d, w2, b2)


_MESH = plsc.VectorSubcoreMesh(core_axis_name="c", subcore_axis_name="s",
                               num_cores=2, num_subcores=16)
NTILE = 16
ECHUNK = 128
ROWS_PER_TILE = NPAD // NTILE

_ECH = 128
_INNER = 16
_BLOCKS = 10
_AGG_CHUNKS = _INNER * _BLOCKS
EPAD = _AGG_CHUNKS * _ECH * NTILE
_NPADROWS = NPAD - N


def _zero_vmem_2d(ref, nrows):
    def zrow(i, _):
        for j in range(HH // 16):
            ref[i, pl.ds(j * 16, 16)] = jnp.zeros((16,), jnp.float32)
        return 0
    lax.fori_loop(0, nrows, zrow, 0)


_CNT_ROWS = EPAD // _ECH // 32


def _count_body(col2d, out, cnt_sh, z640, idx_v, ones_v):
    c = lax.axis_index("c")
    s = lax.axis_index("s")
    for j in range(640 // 16):
        z640[pl.ds(j * 16, 16)] = jnp.zeros((16,), jnp.float32)
    for j in range(_ECH // 16):
        ones_v[pl.ds(j * 16, 16)] = jnp.ones((16,), jnp.float32)
    pltpu.sync_copy(z640, cnt_sh.at[pl.ds(s * ROWS_PER_TILE, ROWS_PER_TILE)])
    w = c * NTILE + s
    pltpu.sync_copy(col2d.at[pl.ds(w * _CNT_ROWS, _CNT_ROWS)], idx_v)
    plsc.subcore_barrier()

    def step(i, _):
        pltpu.sync_copy(ones_v, cnt_sh.at[idx_v.at[i]], add=True)
        return 0

    lax.fori_loop(0, _CNT_ROWS, step, 0)
    plsc.subcore_barrier()
    sl = pl.ds(s * ROWS_PER_TILE, ROWS_PER_TILE)
    pltpu.sync_copy(cnt_sh.at[sl], z640)
    pltpu.sync_copy(z640, out.at[c, sl])


def _count_stage(col2d):
    out = pl.kernel(
        _count_body,
        out_type=jax.ShapeDtypeStruct((2, NPAD), jnp.float32),
        mesh=_MESH,
        scratch_types=[
            pltpu.VMEM_SHARED((NPAD,), jnp.float32),
            pltpu.VMEM((ROWS_PER_TILE,), jnp.float32),
            pltpu.VMEM((_CNT_ROWS, _ECH), jnp.int32),
            pltpu.VMEM((_ECH,), jnp.float32),
        ],
    )(col2d)
    return out[0].reshape(NPAD, 1), out[1].reshape(NPAD, 1)


def _agg_body(ht0, ht1, row2d, col2d, s0, s1, agg_sh, row_v, col_v,
              buf_a, buf_b, ga, gb):
    c = lax.axis_index("c")
    s = lax.axis_index("s")

    def fire_g(k, buf, sem):
        @pl.when(c == 0)
        def _():
            pltpu.async_copy(ht0.at[row_v.at[k]], buf, sem)

        @pl.when(c == 1)
        def _():
            pltpu.async_copy(ht1.at[row_v.at[k]], buf, sem)

    def wait_g(buf, sem):
        pltpu.make_async_copy(ht0.at[pl.ds(0, _ECH)], buf, sem).wait()

    def scat(k, buf):
        pltpu.sync_copy(buf, agg_sh.at[col_v.at[k]], add=True)

    _zero_vmem_2d(buf_a, _ECH)
    for k in range(ROWS_PER_TILE // _ECH):
        pltpu.sync_copy(
            buf_a, agg_sh.at[pl.ds(s * ROWS_PER_TILE + k * _ECH, _ECH)])
    plsc.subcore_barrier()

    def block(o, _):
        obase = s * _AGG_CHUNKS + o * _INNER
        pltpu.sync_copy(row2d.at[pl.ds(obase, _INNER)], row_v)
        pltpu.sync_copy(col2d.at[pl.ds(obase, _INNER)], col_v)
        fire_g(0, buf_a, ga)

        def step(j, _):
            k0 = 2 * j
            fire_g(k0 + 1, buf_b, gb)
            wait_g(buf_a, ga)
            scat(k0, buf_a)

            @pl.when(k0 + 2 < _INNER)
            def _():
                fire_g(k0 + 2, buf_a, ga)

            wait_g(buf_b, gb)
            scat(k0 + 1, buf_b)
            return 0

        lax.fori_loop(0, _INNER // 2, step, 0)
        return 0

    lax.fori_loop(0, _BLOCKS, block, 0)
    plsc.subcore_barrier()

    for k in range(ROWS_PER_TILE // _ECH):
        sl = pl.ds(s * ROWS_PER_TILE + k * _ECH, _ECH)
        pltpu.sync_copy(agg_sh.at[sl], buf_a)

        @pl.when(c == 0)
        def _():
            pltpu.sync_copy(buf_a, s0.at[sl])

        @pl.when(c == 1)
        def _():
            pltpu.sync_copy(buf_a, s1.at[sl])


def _agg_stage(ht0, ht1, row2d, col2d):
    return pl.kernel(
        _agg_body,
        out_type=[jax.ShapeDtypeStruct((NPAD, HH), jnp.float32),
                  jax.ShapeDtypeStruct((NPAD, HH), jnp.float32)],
        mesh=_MESH,
        scratch_types=[
            pltpu.VMEM_SHARED((NPAD, HH), jnp.float32),
            pltpu.VMEM((_INNER, _ECH), jnp.int32),
            pltpu.VMEM((_INNER, _ECH), jnp.int32),
            pltpu.VMEM((_ECH, HH), jnp.float32),
            pltpu.VMEM((_ECH, HH), jnp.float32),
            pltpu.SemaphoreType.DMA,
            pltpu.SemaphoreType.DMA,
        ],
    )(ht0, ht1, row2d, col2d)


def kernel(x, edge_index, batch, lin1_w, lin1_b, conv_w0, conv_w1, conv_w2,
           lin2_w, lin2_b):
    row = edge_index[0]
    col = edge_index[1]
    pad_idx = (N + jnp.arange(EPAD - E, dtype=jnp.int32) % _NPADROWS)
    row2d = jnp.concatenate([row, pad_idx]).reshape(EPAD // _ECH, _ECH)
    col2d = jnp.concatenate([col, pad_idx]).reshape(EPAD // _ECH, _ECH)
    x_pad = jnp.zeros((NPAD, D_IN), jnp.float32).at[:N].set(x)
    batch2d = jnp.full((NPAD, 1), G, jnp.int32).at[:N, 0].set(batch)
    b1 = lin1_b.reshape(1, H)
    b2 = lin2_b.reshape(1, C)

    cnt0, cnt1 = _count_stage(col2d)
    x0, ht0, ht1, dinv = _lin1_stage(cnt0, cnt1, x_pad, lin1_w, b1)

    betas = [float(np.log(THETA / l + 1.0)) for l in (1, 2, 3)]
    for li, (w, beta) in enumerate(zip((conv_w0, conv_w1, conv_w2), betas)):
        s0, s1 = _agg_stage(ht0, ht1, row2d, col2d)
        if li < 2:
            ht0, ht1 = _layer_stage(s0, s1, ht0, ht1, x0, dinv, w, beta)
        else:
            return _layer3_pool_stage(s0, s1, ht0, ht1, x0, dinv, w, beta,
                                      batch2d, lin2_w, b2)


# speedup vs baseline: 1.3733x; 1.0401x over previous
import functools
import numpy as np

import jax
import jax.numpy as jnp
from jax import lax
from jax.experimental import pallas as pl
from jax.experimental.pallas import tpu as pltpu
from jax.experimental.pallas import tpu_sc as plsc

N = 10000
NPAD = 10240
E = 320000
D_IN = 128
H = 256
HH = 128
C = 32
G = 128
ALPHA = 0.5
THETA = 0.1
BLK = 512
NBLK = NPAD // BLK


def _lin1_body(cnt0, cnt1, x, w1, b1, x0_out, ht0_out, ht1_out, dinv_out):
    deg = cnt0[...] + cnt1[...] + 1.0
    dinv = lax.rsqrt(deg)
    h = jnp.dot(x[...], w1[...], preferred_element_type=jnp.float32) + b1[...]
    h = jnp.maximum(h, 0.0)
    x0_out[...] = h
    ht = dinv * h
    ht0_out[...] = ht[:, :HH]
    ht1_out[...] = ht[:, HH:]
    dinv_out[...] = dinv


def _lin1_stage(cnt0, cnt1, x_pad, w1, b1):
    return pl.pallas_call(
        _lin1_body,
        grid=(NBLK,),
        in_specs=[
            pl.BlockSpec((BLK, 1), lambda i: (i, 0)),
            pl.BlockSpec((BLK, 1), lambda i: (i, 0)),
            pl.BlockSpec((BLK, D_IN), lambda i: (i, 0)),
            pl.BlockSpec((D_IN, H), lambda i: (0, 0)),
            pl.BlockSpec((1, H), lambda i: (0, 0)),
        ],
        out_specs=[
            pl.BlockSpec((BLK, H), lambda i: (i, 0)),
            pl.BlockSpec((BLK, HH), lambda i: (i, 0)),
            pl.BlockSpec((BLK, HH), lambda i: (i, 0)),
            pl.BlockSpec((BLK, 1), lambda i: (i, 0)),
        ],
        out_shape=[
            jax.ShapeDtypeStruct((NPAD, H), jnp.float32),
            jax.ShapeDtypeStruct((NPAD, HH), jnp.float32),
            jax.ShapeDtypeStruct((NPAD, HH), jnp.float32),
            jax.ShapeDtypeStruct((NPAD, 1), jnp.float32),
        ],
    )(cnt0, cnt1, x_pad, w1, b1)


def _layer_h(s0, s1, ht0, ht1, x0, dinv, w, beta):
    sfull = jnp.concatenate([s0[...] + ht0[...], s1[...] + ht1[...]], axis=1)
    agg = dinv[...] * sfull
    out = (1.0 - ALPHA) * agg + ALPHA * x0[...]
    z = (1.0 - beta) * out + beta * jnp.dot(
        out, w[...], preferred_element_type=jnp.float32)
    return jnp.maximum(z, 0.0)


def _layer_body(s0, s1, ht0, ht1, x0, dinv, w, ht0_out, ht1_out, *, beta):
    h = _layer_h(s0, s1, ht0, ht1, x0, dinv, w, beta)
    ht = dinv[...] * h
    ht0_out[...] = ht[:, :HH]
    ht1_out[...] = ht[:, HH:]


def _layer_stage(s0, s1, ht0, ht1, x0, dinv, w, beta):
    half = pl.BlockSpec((BLK, HH), lambda i: (i, 0))
    return pl.pallas_call(
        functools.partial(_layer_body, beta=beta),
        grid=(NBLK,),
        in_specs=[half, half, half, half,
                  pl.BlockSpec((BLK, H), lambda i: (i, 0)),
                  pl.BlockSpec((BLK, 1), lambda i: (i, 0)),
                  pl.BlockSpec((H, H), lambda i: (0, 0))],
        out_specs=[half, half],
        out_shape=[jax.ShapeDtypeStruct((NPAD, HH), jnp.float32)] * 2,
    )(s0, s1, ht0, ht1, x0, dinv, w)


def _layer3_pool_body(s0, s1, ht0, ht1, x0, dinv, w, batch, w2, b2, out,
                      acc, cnt, *, beta):
    i = pl.program_id(0)

    @pl.when(i == 0)
    def _():
        acc[...] = jnp.zeros_like(acc)
        cnt[...] = jnp.zeros_like(cnt)

    h = _layer_h(s0, s1, ht0, ht1, x0, dinv, w, beta)
    gid = lax.broadcasted_iota(jnp.int32, (BLK, G), 1)
    p = (batch[...] == gid).astype(jnp.float32)
    dn = (((0,), (0,)), ((), ()))
    acc[...] += lax.dot_general(p, h, dn, preferred_element_type=jnp.float32)
    cnt[...] += lax.dot_general(p, jnp.ones((BLK, 1), jnp.float32), dn,
                                preferred_element_type=jnp.float32)

    @pl.when(i == NBLK - 1)
    def _():
        pooled = acc[...] / jnp.maximum(cnt[...], 1.0)
        out[...] = jnp.dot(pooled, w2[...],
                           preferred_element_type=jnp.float32) + b2[...]


def _layer3_pool_stage(s0, s1, ht0, ht1, x0, dinv, w, beta, batch2d, w2, b2):
    half = pl.BlockSpec((BLK, HH), lambda i: (i, 0))
    return pl.pallas_call(
        functools.partial(_layer3_pool_body, beta=beta),
        grid=(NBLK,),
        in_specs=[half, half, half, half,
                  pl.BlockSpec((BLK, H), lambda i: (i, 0)),
                  pl.BlockSpec((BLK, 1), lambda i: (i, 0)),
                  pl.BlockSpec((H, H), lambda i: (0, 0)),
                  pl.BlockSpec((BLK, 1), lambda i: (i, 0)),
                  pl.BlockSpec((H, C), lambda i: (0, 0)),
                  pl.BlockSpec((1, C), lambda i: (0, 0))],
        out_specs=pl.BlockSpec((G, C), lambda i: (0, 0)),
        out_shape=jax.ShapeDtypeStruct((G, C), jnp.float32),
        scratch_shapes=[pltpu.VMEM((G, H), jnp.float32),
                        pltpu.VMEM((G, 1), jnp.float32)],
    )(s0, s1, ht0, ht1, x0, dinv, w, batch2d, w2, b2)


_MESH = plsc.VectorSubcoreMesh(core_axis_name="c", subcore_axis_name="s",
                               num_cores=2, num_subcores=16)
NTILE = 16
ECHUNK = 128
ROWS_PER_TILE = NPAD // NTILE

_ECH = 128
_INNER = 16
_BLOCKS = 10
_AGG_CHUNKS = _INNER * _BLOCKS
EPAD = _AGG_CHUNKS * _ECH * NTILE
_NPADROWS = NPAD - N


def _zero_vmem_2d(ref, nrows):
    def zrow(i, _):
        for j in range(HH // 16):
            ref[i, pl.ds(j * 16, 16)] = jnp.zeros((16,), jnp.float32)
        return 0
    lax.fori_loop(0, nrows, zrow, 0)


_CNT_ROWS = EPAD // _ECH // 32


def _count_body(col2d, out, cnt_sh, z640, idx_v, ones_v):
    c = lax.axis_index("c")
    s = lax.axis_index("s")
    for j in range(640 // 16):
        z640[pl.ds(j * 16, 16)] = jnp.zeros((16,), jnp.float32)
    for j in range(_ECH // 16):
        ones_v[pl.ds(j * 16, 16)] = jnp.ones((16,), jnp.float32)
    pltpu.sync_copy(z640, cnt_sh.at[pl.ds(s * ROWS_PER_TILE, ROWS_PER_TILE)])
    w = c * NTILE + s
    pltpu.sync_copy(col2d.at[pl.ds(w * _CNT_ROWS, _CNT_ROWS)], idx_v)
    plsc.subcore_barrier()

    def step(i, _):
        pltpu.sync_copy(ones_v, cnt_sh.at[idx_v.at[i]], add=True)
        return 0

    lax.fori_loop(0, _CNT_ROWS, step, 0)
    plsc.subcore_barrier()
    sl = pl.ds(s * ROWS_PER_TILE, ROWS_PER_TILE)
    pltpu.sync_copy(cnt_sh.at[sl], z640)
    pltpu.sync_copy(z640, out.at[c, sl])


def _count_stage(col2d):
    out = pl.kernel(
        _count_body,
        out_type=jax.ShapeDtypeStruct((2, NPAD), jnp.float32),
        mesh=_MESH,
        scratch_types=[
            pltpu.VMEM_SHARED((NPAD,), jnp.float32),
            pltpu.VMEM((ROWS_PER_TILE,), jnp.float32),
            pltpu.VMEM((_CNT_ROWS, _ECH), jnp.int32),
            pltpu.VMEM((_ECH,), jnp.float32),
        ],
    )(col2d)
    return out[0].reshape(NPAD, 1), out[1].reshape(NPAD, 1)


def _agg_body(ht0, ht1, row2d, col2d, s0, s1, agg_sh, row_va, col_va,
              row_vb, col_vb, buf_a, buf_b, ga, gb, ia, ib):
    c = lax.axis_index("c")
    s = lax.axis_index("s")

    def fire_g(row_v, k, buf, sem):
        @pl.when(c == 0)
        def _():
            pltpu.async_copy(ht0.at[row_v.at[k]], buf, sem)

        @pl.when(c == 1)
        def _():
            pltpu.async_copy(ht1.at[row_v.at[k]], buf, sem)

    def wait_g(buf, sem):
        pltpu.make_async_copy(ht0.at[pl.ds(0, _ECH)], buf, sem).wait()

    def fire_idx(o, row_v, col_v, sem):
        obase = s * _AGG_CHUNKS + o * _INNER
        pltpu.async_copy(row2d.at[pl.ds(obase, _INNER)], row_v, sem)
        pltpu.async_copy(col2d.at[pl.ds(obase, _INNER)], col_v, sem)

    def wait_idx(row_v, col_v, sem):
        pltpu.make_async_copy(row2d.at[pl.ds(0, _INNER)], row_v, sem).wait()
        pltpu.make_async_copy(col2d.at[pl.ds(0, _INNER)], col_v, sem).wait()

    def process_block(row_v, col_v, buf_sems):
        fire_g(row_v, 0, buf_a, ga)

        def step(j, _):
            k0 = 2 * j
            fire_g(row_v, k0 + 1, buf_b, gb)
            wait_g(buf_a, ga)
            pltpu.sync_copy(buf_a, agg_sh.at[col_v.at[k0]], add=True)

            @pl.when(k0 + 2 < _INNER)
            def _():
                fire_g(row_v, k0 + 2, buf_a, ga)

            wait_g(buf_b, gb)
            pltpu.sync_copy(buf_b, agg_sh.at[col_v.at[k0 + 1]], add=True)
            return 0

        lax.fori_loop(0, _INNER // 2, step, 0)

    _zero_vmem_2d(buf_a, _ECH)
    for k in range(ROWS_PER_TILE // _ECH):
        pltpu.sync_copy(
            buf_a, agg_sh.at[pl.ds(s * ROWS_PER_TILE + k * _ECH, _ECH)])
    fire_idx(0, row_va, col_va, ia)
    plsc.subcore_barrier()

    def pair(m, _):
        o0 = 2 * m
        wait_idx(row_va, col_va, ia)
        fire_idx(o0 + 1, row_vb, col_vb, ib)
        process_block(row_va, col_va, None)
        wait_idx(row_vb, col_vb, ib)

        @pl.when(o0 + 2 < _BLOCKS)
        def _():
            fire_idx(o0 + 2, row_va, col_va, ia)

        process_block(row_vb, col_vb, None)
        return 0

    lax.fori_loop(0, _BLOCKS // 2, pair, 0)
    plsc.subcore_barrier()

    for k in range(ROWS_PER_TILE // _ECH):
        sl = pl.ds(s * ROWS_PER_TILE + k * _ECH, _ECH)
        pltpu.sync_copy(agg_sh.at[sl], buf_a)

        @pl.when(c == 0)
        def _():
            pltpu.sync_copy(buf_a, s0.at[sl])

        @pl.when(c == 1)
        def _():
            pltpu.sync_copy(buf_a, s1.at[sl])


def _agg_stage(ht0, ht1, row2d, col2d):
    return pl.kernel(
        _agg_body,
        out_type=[jax.ShapeDtypeStruct((NPAD, HH), jnp.float32),
                  jax.ShapeDtypeStruct((NPAD, HH), jnp.float32)],
        mesh=_MESH,
        scratch_types=[
            pltpu.VMEM_SHARED((NPAD, HH), jnp.float32),
            pltpu.VMEM((_INNER, _ECH), jnp.int32),
            pltpu.VMEM((_INNER, _ECH), jnp.int32),
            pltpu.VMEM((_INNER, _ECH), jnp.int32),
            pltpu.VMEM((_INNER, _ECH), jnp.int32),
            pltpu.VMEM((_ECH, HH), jnp.float32),
            pltpu.VMEM((_ECH, HH), jnp.float32),
            pltpu.SemaphoreType.DMA,
            pltpu.SemaphoreType.DMA,
            pltpu.SemaphoreType.DMA,
            pltpu.SemaphoreType.DMA,
        ],
    )(ht0, ht1, row2d, col2d)


def kernel(x, edge_index, batch, lin1_w, lin1_b, conv_w0, conv_w1, conv_w2,
           lin2_w, lin2_b):
    row = edge_index[0]
    col = edge_index[1]
    pad_idx = (N + jnp.arange(EPAD - E, dtype=jnp.int32) % _NPADROWS)
    row2d = jnp.concatenate([row, pad_idx]).reshape(EPAD // _ECH, _ECH)
    col2d = jnp.concatenate([col, pad_idx]).reshape(EPAD // _ECH, _ECH)
    x_pad = jnp.zeros((NPAD, D_IN), jnp.float32).at[:N].set(x)
    batch2d = jnp.full((NPAD, 1), G, jnp.int32).at[:N, 0].set(batch)
    b1 = lin1_b.reshape(1, H)
    b2 = lin2_b.reshape(1, C)

    cnt0, cnt1 = _count_stage(col2d)
    x0, ht0, ht1, dinv = _lin1_stage(cnt0, cnt1, x_pad, lin1_w, b1)

    betas = [float(np.log(THETA / l + 1.0)) for l in (1, 2, 3)]
    for li, (w, beta) in enumerate(zip((conv_w0, conv_w1, conv_w2), betas)):
        s0, s1 = _agg_stage(ht0, ht1, row2d, col2d)
        if li < 2:
            ht0, ht1 = _layer_stage(s0, s1, ht0, ht1, x0, dinv, w, beta)
        else:
            return _layer3_pool_stage(s0, s1, ht0, ht1, x0, dinv, w, beta,
                                      batch2d, lin2_w, b2)
